# baseline (speedup 1.0000x reference)
import jax
import jax.numpy as jnp
from jax import lax
from jax.experimental import pallas as pl
from jax.experimental.pallas import tpu as pltpu
from jax.experimental.pallas import tpu_sc as plsc

B = 16384
ED = 64
CF = 64
PF = 32
NW = 32
BPW = B // NW
H = BPW // 2


def _sc_gather(ct2, pt2, cid_hbm, pid_hbm, ce2_hbm, pe2_hbm,
               cidx_v, pidx_v, cbuf, pbuf, sem_c, sem_p):
    wid = lax.axis_index("s") * 2 + lax.axis_index("c")
    base = wid * BPW
    pltpu.sync_copy(cid_hbm.at[pl.ds(base, BPW)], cidx_v)
    pltpu.sync_copy(pid_hbm.at[pl.ds(base, BPW)], pidx_v)
    for h in range(2):
        cc = pltpu.async_copy(ct2.at[cidx_v.at[pl.ds(h * H, H)]], cbuf, sem_c)
        pc = pltpu.async_copy(pt2.at[pidx_v.at[pl.ds(h * H, H)]], pbuf, sem_p)
        cc.wait()
        pltpu.sync_copy(cbuf, ce2_hbm.at[pl.ds(base + h * H, H)])
        pc.wait()
        pltpu.sync_copy(pbuf, pe2_hbm.at[pl.ds(base + h * H, H)])


def _mlp_body(ce2, pe2, cpar, ppar, cf, pf,
              w1c, w1p, w1cf, w1pf, b1, w2, b2, w3, b3, w4, b4, out_ref):
    ce = jnp.where(cpar[...] > 0, ce2[:, ED:], ce2[:, :ED])
    pe = jnp.where(ppar[...] > 0, pe2[:, ED:], pe2[:, :ED])
    x = jnp.dot(ce, w1c[...], preferred_element_type=jnp.float32)
    x += jnp.dot(pe, w1p[...], preferred_element_type=jnp.float32)
    x += jnp.dot(cf[...], w1cf[...], preferred_element_type=jnp.float32)
    x += jnp.dot(pf[...], w1pf[...], preferred_element_type=jnp.float32)
    h = jax.nn.relu(x + b1[...])
    h = jax.nn.relu(jnp.dot(h, w2[...], preferred_element_type=jnp.float32)
                    + b2[...])
    h = jax.nn.relu(jnp.dot(h, w3[...], preferred_element_type=jnp.float32)
                    + b3[...])
    o = jnp.dot(h, w4[...], preferred_element_type=jnp.float32) + b4[...]
    out_ref[...] = jax.nn.sigmoid(o)


def kernel(customer_ids, product_ids, customer_features, product_features,
           customer_table, product_table,
           W1, b1, g1, beta1, W2, b2, g2, beta2, W3, b3, g3, beta3, W4, b4):
    cid = customer_ids.astype(jnp.int32)
    pid = product_ids.astype(jnp.int32)
    nc = customer_table.shape[0]
    np_ = product_table.shape[0]
    ct2 = customer_table.reshape(nc // 2, 2 * ED)
    pt2 = product_table.reshape(np_ // 2, 2 * ED)

    mesh = plsc.VectorSubcoreMesh(core_axis_name="c", subcore_axis_name="s")
    gather = pl.kernel(
        _sc_gather,
        out_type=(jax.ShapeDtypeStruct((B, 2 * ED), jnp.float32),
                  jax.ShapeDtypeStruct((B, 2 * ED), jnp.float32)),
        mesh=mesh,
        scratch_types=[
            pltpu.VMEM((BPW,), jnp.int32),
            pltpu.VMEM((BPW,), jnp.int32),
            pltpu.VMEM((H, 2 * ED), jnp.float32),
            pltpu.VMEM((H, 2 * ED), jnp.float32),
            pltpu.SemaphoreType.DMA,
            pltpu.SemaphoreType.DMA,
        ],
    )
    ce2, pe2 = gather(ct2, pt2, cid // 2, pid // 2)
    cpar = (cid % 2).reshape(B, 1)
    ppar = (pid % 2).reshape(B, 1)

    inv = 1.0 / jnp.sqrt(1.0 + 1e-5)
    s1 = g1 * inv
    s2 = g2 * inv
    s3 = g3 * inv
    w1f = W1 * s1[:, None]
    b1f = (b1 * s1 + beta1).reshape(1, -1)
    w2f = (W2 * s2[:, None]).T
    b2f = (b2 * s2 + beta2).reshape(1, -1)
    w3f = (W3 * s3[:, None]).T
    b3f = (b3 * s3 + beta3).reshape(1, -1)
    w4t = W4.T
    b4r = b4.reshape(1, -1)
    w1c = w1f[:, :ED].T
    w1p = w1f[:, ED:2 * ED].T
    w1cf = w1f[:, 2 * ED:2 * ED + CF].T
    w1pf = w1f[:, 2 * ED + CF:].T

    BM = 2048
    grid = B // BM
    row = lambda i: (i, 0)
    full = lambda i: (0, 0)
    out = pl.pallas_call(
        _mlp_body,
        grid=(grid,),
        in_specs=[
            pl.BlockSpec((BM, 2 * ED), row),
            pl.BlockSpec((BM, 2 * ED), row),
            pl.BlockSpec((BM, 1), row),
            pl.BlockSpec((BM, 1), row),
            pl.BlockSpec((BM, CF), row),
            pl.BlockSpec((BM, PF), row),
            pl.BlockSpec((ED, 256), full),
            pl.BlockSpec((ED, 256), full),
            pl.BlockSpec((CF, 256), full),
            pl.BlockSpec((PF, 256), full),
            pl.BlockSpec((1, 256), full),
            pl.BlockSpec((256, 128), full),
            pl.BlockSpec((1, 128), full),
            pl.BlockSpec((128, 64), full),
            pl.BlockSpec((1, 64), full),
            pl.BlockSpec((64, 1), full),
            pl.BlockSpec((1, 1), full),
        ],
        out_specs=pl.BlockSpec((BM, 1), row),
        out_shape=jax.ShapeDtypeStruct((B, 1), jnp.float32),
    )(ce2, pe2, cpar, ppar, customer_features, product_features,
      w1c, w1p, w1cf, w1pf, b1f, w2f, b2f, w3f, b3f, w4t, b4r)
    return out


# speedup vs baseline: 1.6025x; 1.6025x over previous
import jax
import jax.numpy as jnp
from jax import lax
from jax.experimental import pallas as pl
from jax.experimental.pallas import tpu as pltpu
from jax.experimental.pallas import tpu_sc as plsc

B = 16384
ED = 64
CF = 64
PF = 32
NW = 32
BPW = B // NW
H = BPW // 2


def _sc_gather(ct4, pt4, cid_hbm, pid_hbm, ce4_hbm, pe4_hbm,
               cidx_v, pidx_v, cbuf, pbuf, sem_c, sem_p):
    wid = lax.axis_index("s") * 2 + lax.axis_index("c")
    base = wid * BPW
    pltpu.sync_copy(cid_hbm.at[pl.ds(base, BPW)], cidx_v)
    pltpu.sync_copy(pid_hbm.at[pl.ds(base, BPW)], pidx_v)
    for h in range(2):
        cc = pltpu.async_copy(ct4.at[cidx_v.at[pl.ds(h * H, H)]], cbuf, sem_c)
        pc = pltpu.async_copy(pt4.at[pidx_v.at[pl.ds(h * H, H)]], pbuf, sem_p)
        cc.wait()
        pltpu.sync_copy(cbuf, ce4_hbm.at[pl.ds(base + h * H, H)])
        pc.wait()
        pltpu.sync_copy(pbuf, pe4_hbm.at[pl.ds(base + h * H, H)])


def _cvt_body(xT_ref, out_ref):
    u = lax.bitcast_convert_type(xT_ref[...], jnp.uint32)
    r = (u + jnp.uint32(0x7FFF) + ((u >> 16) & jnp.uint32(1))) >> 16
    w = r[:32, :] | (r[32:, :] << 16)
    parts = [jnp.transpose(w[:, j * 1024:(j + 1) * 1024]) for j in range(4)]
    out_ref[...] = lax.bitcast_convert_type(
        jnp.concatenate(parts, axis=1), jnp.int32)


def _pack_table(tbl):
    n = tbl.shape[0]
    bk = 4096
    grid = pl.cdiv(n, bk)
    return pl.pallas_call(
        _cvt_body,
        grid=(grid,),
        in_specs=[pl.BlockSpec((ED, bk), lambda i: (0, i))],
        out_specs=pl.BlockSpec((bk // 4, 128), lambda i: (i, 0)),
        out_shape=jax.ShapeDtypeStruct((grid * (bk // 4), 128), jnp.int32),
    )(tbl.T)


def _pick(e32, m):
    lo = lax.bitcast_convert_type(e32 << 16, jnp.float32)
    hi = lax.bitcast_convert_type(e32 & jnp.int32(-65536), jnp.float32)

    def sel(x):
        a = jnp.where(m < 1, x[:, :32], x[:, 32:64])
        b = jnp.where(m < 3, x[:, 64:96], x[:, 96:])
        return jnp.where(m < 2, a, b)

    return sel(lo), sel(hi)


def _mlp_body(ce4, pe4, cm, pm, cf, pf, w1ce, w1co, w1pe, w1po,
              w1cf, w1pf, b1, w2, b2, w3, b3, w4, b4, out_ref):
    ce_e, ce_o = _pick(ce4[...], cm[...])
    pe_e, pe_o = _pick(pe4[...], pm[...])
    x = jnp.dot(ce_e, w1ce[...], preferred_element_type=jnp.float32)
    x += jnp.dot(ce_o, w1co[...], preferred_element_type=jnp.float32)
    x += jnp.dot(pe_e, w1pe[...], preferred_element_type=jnp.float32)
    x += jnp.dot(pe_o, w1po[...], preferred_element_type=jnp.float32)
    x += jnp.dot(cf[...], w1cf[...], preferred_element_type=jnp.float32)
    x += jnp.dot(pf[...], w1pf[...], preferred_element_type=jnp.float32)
    h = jax.nn.relu(x + b1[...])
    h = jax.nn.relu(jnp.dot(h, w2[...], preferred_element_type=jnp.float32)
                    + b2[...])
    h = jax.nn.relu(jnp.dot(h, w3[...], preferred_element_type=jnp.float32)
                    + b3[...])
    o = jnp.dot(h, w4[...], preferred_element_type=jnp.float32) + b4[...]
    out_ref[...] = jax.nn.sigmoid(o)


def kernel(customer_ids, product_ids, customer_features, product_features,
           customer_table, product_table,
           W1, b1, g1, beta1, W2, b2, g2, beta2, W3, b3, g3, beta3, W4, b4):
    cid = customer_ids.astype(jnp.int32)
    pid = product_ids.astype(jnp.int32)
    nc = customer_table.shape[0]
    np_ = product_table.shape[0]
    ct4 = _pack_table(customer_table)
    pt4 = _pack_table(product_table)

    mesh = plsc.VectorSubcoreMesh(core_axis_name="c", subcore_axis_name="s")
    gather = pl.kernel(
        _sc_gather,
        out_type=(jax.ShapeDtypeStruct((B, 128), jnp.int32),
                  jax.ShapeDtypeStruct((B, 128), jnp.int32)),
        mesh=mesh,
        scratch_types=[
            pltpu.VMEM((BPW,), jnp.int32),
            pltpu.VMEM((BPW,), jnp.int32),
            pltpu.VMEM((H, 128), jnp.int32),
            pltpu.VMEM((H, 128), jnp.int32),
            pltpu.SemaphoreType.DMA,
            pltpu.SemaphoreType.DMA,
        ],
    )
    cu = (cid // 4096) * 1024 + cid % 1024
    pu = (pid // 4096) * 1024 + pid % 1024
    ce4, pe4 = gather(ct4, pt4, cu, pu)
    cm = ((cid % 4096) // 1024).reshape(B, 1)
    pm = ((pid % 4096) // 1024).reshape(B, 1)

    inv = 1.0 / jnp.sqrt(1.0 + 1e-5)
    s1 = g1 * inv
    s2 = g2 * inv
    s3 = g3 * inv
    w1f = W1 * s1[:, None]
    b1f = (b1 * s1 + beta1).reshape(1, -1)
    w2f = (W2 * s2[:, None]).T
    b2f = (b2 * s2 + beta2).reshape(1, -1)
    w3f = (W3 * s3[:, None]).T
    b3f = (b3 * s3 + beta3).reshape(1, -1)
    w4t = W4.T
    b4r = b4.reshape(1, 1)
    w1c = w1f[:, :ED].T
    w1p = w1f[:, ED:2 * ED].T
    w1cf = w1f[:, 2 * ED:2 * ED + CF].T
    w1pf = w1f[:, 2 * ED + CF:].T
    w1ce, w1co = w1c[:32], w1c[32:]
    w1pe, w1po = w1p[:32], w1p[32:]

    BM = 2048
    grid = B // BM
    row = lambda i: (i, 0)
    row3 = lambda i: (i, 0, 0)
    full = lambda i: (0, 0)
    out = pl.pallas_call(
        _mlp_body,
        grid=(grid,),
        in_specs=[
            pl.BlockSpec((BM, 128), row),
            pl.BlockSpec((BM, 128), row),
            pl.BlockSpec((BM, 1), row),
            pl.BlockSpec((BM, 1), row),
            pl.BlockSpec((BM, CF), row),
            pl.BlockSpec((BM, PF), row),
            pl.BlockSpec((32, 256), full),
            pl.BlockSpec((32, 256), full),
            pl.BlockSpec((32, 256), full),
            pl.BlockSpec((32, 256), full),
            pl.BlockSpec((CF, 256), full),
            pl.BlockSpec((PF, 256), full),
            pl.BlockSpec((1, 256), full),
            pl.BlockSpec((256, 128), full),
            pl.BlockSpec((1, 128), full),
            pl.BlockSpec((128, 64), full),
            pl.BlockSpec((1, 64), full),
            pl.BlockSpec((64, 1), full),
            pl.BlockSpec((1, 1), full),
        ],
        out_specs=pl.BlockSpec((BM, 1), row),
        out_shape=jax.ShapeDtypeStruct((B, 1), jnp.float32),
    )(ce4, pe4, cm, pm, customer_features, product_features,
      w1ce, w1co, w1pe, w1po, w1cf, w1pf, b1f, w2f, b2f, w3f, b3f, w4t, b4r)
    return out


# speedup vs baseline: 1.8862x; 1.1770x over previous
import jax
import jax.numpy as jnp
from jax import lax
from jax.experimental import pallas as pl
from jax.experimental.pallas import tpu as pltpu
from jax.experimental.pallas import tpu_sc as plsc

B = 16384
ED = 64
CF = 64
PF = 32
NW = 32
BPW = B // NW
H = BPW // 2


def _sc_gather(ct4, pt4, cid_hbm, pid_hbm, ce4_hbm, pe4_hbm,
               cidx_v, pidx_v, cbuf, pbuf, sem_c, sem_p):
    wid = lax.axis_index("s") * 2 + lax.axis_index("c")
    base = wid * BPW
    pltpu.sync_copy(cid_hbm.at[pl.ds(base, BPW)], cidx_v)
    pltpu.sync_copy(pid_hbm.at[pl.ds(base, BPW)], pidx_v)
    for h in range(2):
        cc = pltpu.async_copy(ct4.at[cidx_v.at[pl.ds(h * H, H)]], cbuf, sem_c)
        pc = pltpu.async_copy(pt4.at[pidx_v.at[pl.ds(h * H, H)]], pbuf, sem_p)
        cc.wait()
        pltpu.sync_copy(cbuf, ce4_hbm.at[pl.ds(base + h * H, H)])
        pc.wait()
        pltpu.sync_copy(pbuf, pe4_hbm.at[pl.ds(base + h * H, H)])


def _cvt_body(xT_ref, out_ref):
    u = lax.bitcast_convert_type(xT_ref[...], jnp.uint32)
    r = (u + jnp.uint32(0x7FFF) + ((u >> 16) & jnp.uint32(1))) >> 16
    w = r[:32, :] | (r[32:, :] << 16)
    q = w.shape[1] // 4
    parts = [jnp.transpose(w[:, j * q:(j + 1) * q]) for j in range(4)]
    out_ref[...] = lax.bitcast_convert_type(
        jnp.concatenate(parts, axis=1), jnp.int32)


def _pack_table(tbl):
    n = tbl.shape[0]
    bk = 8192
    grid = pl.cdiv(n, bk)
    return pl.pallas_call(
        _cvt_body,
        grid=(grid,),
        in_specs=[pl.BlockSpec((ED, bk), lambda i: (0, i))],
        out_specs=pl.BlockSpec((bk // 4, 128), lambda i: (i, 0)),
        out_shape=jax.ShapeDtypeStruct((grid * (bk // 4), 128), jnp.int32),
    )(tbl.T)


def _pick(e32, m):
    lo = lax.bitcast_convert_type(e32 << 16, jnp.float32)
    hi = lax.bitcast_convert_type(e32 & jnp.int32(-65536), jnp.float32)

    def sel(x):
        a = jnp.where(m < 1, x[:, :32], x[:, 32:64])
        b = jnp.where(m < 3, x[:, 64:96], x[:, 96:])
        return jnp.where(m < 2, a, b)

    return sel(lo), sel(hi)


def _mlp_body(ce4, pe4, cm, pm, cf, pf, w1ce, w1co, w1pe, w1po,
              w1cf, w1pf, b1, w2, b2, w3, b3, w4, b4, out_ref):
    ce_e, ce_o = _pick(ce4[...], cm[...])
    pe_e, pe_o = _pick(pe4[...], pm[...])
    x = jnp.dot(ce_e, w1ce[...], preferred_element_type=jnp.float32)
    x += jnp.dot(ce_o, w1co[...], preferred_element_type=jnp.float32)
    x += jnp.dot(pe_e, w1pe[...], preferred_element_type=jnp.float32)
    x += jnp.dot(pe_o, w1po[...], preferred_element_type=jnp.float32)
    x += jnp.dot(cf[...], w1cf[...], preferred_element_type=jnp.float32)
    x += jnp.dot(pf[...], w1pf[...], preferred_element_type=jnp.float32)
    h = jax.nn.relu(x + b1[...])
    h = jax.nn.relu(jnp.dot(h, w2[...], preferred_element_type=jnp.float32)
                    + b2[...])
    h = jax.nn.relu(jnp.dot(h, w3[...], preferred_element_type=jnp.float32)
                    + b3[...])
    o = jnp.dot(h, w4[...], preferred_element_type=jnp.float32) + b4[...]
    out_ref[...] = jax.nn.sigmoid(o)


def kernel(customer_ids, product_ids, customer_features, product_features,
           customer_table, product_table,
           W1, b1, g1, beta1, W2, b2, g2, beta2, W3, b3, g3, beta3, W4, b4):
    cid = customer_ids.astype(jnp.int32)
    pid = product_ids.astype(jnp.int32)
    nc = customer_table.shape[0]
    np_ = product_table.shape[0]
    ct4 = _pack_table(customer_table)
    pt4 = _pack_table(product_table)

    mesh = plsc.VectorSubcoreMesh(core_axis_name="c", subcore_axis_name="s")
    gather = pl.kernel(
        _sc_gather,
        out_type=(jax.ShapeDtypeStruct((B, 128), jnp.int32),
                  jax.ShapeDtypeStruct((B, 128), jnp.int32)),
        mesh=mesh,
        scratch_types=[
            pltpu.VMEM((BPW,), jnp.int32),
            pltpu.VMEM((BPW,), jnp.int32),
            pltpu.VMEM((H, 128), jnp.int32),
            pltpu.VMEM((H, 128), jnp.int32),
            pltpu.SemaphoreType.DMA,
            pltpu.SemaphoreType.DMA,
        ],
    )
    cu = (cid // 8192) * 2048 + cid % 2048
    pu = (pid // 8192) * 2048 + pid % 2048
    ce4, pe4 = gather(ct4, pt4, cu, pu)
    cm = ((cid % 8192) // 2048).reshape(B, 1)
    pm = ((pid % 8192) // 2048).reshape(B, 1)

    inv = 1.0 / jnp.sqrt(1.0 + 1e-5)
    s1 = g1 * inv
    s2 = g2 * inv
    s3 = g3 * inv
    w1f = W1 * s1[:, None]
    b1f = (b1 * s1 + beta1).reshape(1, -1)
    w2f = (W2 * s2[:, None]).T
    b2f = (b2 * s2 + beta2).reshape(1, -1)
    w3f = (W3 * s3[:, None]).T
    b3f = (b3 * s3 + beta3).reshape(1, -1)
    w4t = W4.T
    b4r = b4.reshape(1, 1)
    w1c = w1f[:, :ED].T
    w1p = w1f[:, ED:2 * ED].T
    w1cf = w1f[:, 2 * ED:2 * ED + CF].T
    w1pf = w1f[:, 2 * ED + CF:].T
    w1ce, w1co = w1c[:32], w1c[32:]
    w1pe, w1po = w1p[:32], w1p[32:]

    BM = 4096
    grid = B // BM
    row = lambda i: (i, 0)
    row3 = lambda i: (i, 0, 0)
    full = lambda i: (0, 0)
    out = pl.pallas_call(
        _mlp_body,
        grid=(grid,),
        in_specs=[
            pl.BlockSpec((BM, 128), row),
            pl.BlockSpec((BM, 128), row),
            pl.BlockSpec((BM, 1), row),
            pl.BlockSpec((BM, 1), row),
            pl.BlockSpec((BM, CF), row),
            pl.BlockSpec((BM, PF), row),
            pl.BlockSpec((32, 256), full),
            pl.BlockSpec((32, 256), full),
            pl.BlockSpec((32, 256), full),
            pl.BlockSpec((32, 256), full),
            pl.BlockSpec((CF, 256), full),
            pl.BlockSpec((PF, 256), full),
            pl.BlockSpec((1, 256), full),
            pl.BlockSpec((256, 128), full),
            pl.BlockSpec((1, 128), full),
            pl.BlockSpec((128, 64), full),
            pl.BlockSpec((1, 64), full),
            pl.BlockSpec((64, 1), full),
            pl.BlockSpec((1, 1), full),
        ],
        out_specs=pl.BlockSpec((BM, 1), row),
        out_shape=jax.ShapeDtypeStruct((B, 1), jnp.float32),
    )(ce4, pe4, cm, pm, customer_features, product_features,
      w1ce, w1co, w1pe, w1po, w1cf, w1pf, b1f, w2f, b2f, w3f, b3f, w4t, b4r)
    return out


# speedup vs baseline: 1.9251x; 1.0206x over previous
import jax
import jax.numpy as jnp
from jax import lax
from jax.experimental import pallas as pl
from jax.experimental.pallas import tpu as pltpu
from jax.experimental.pallas import tpu_sc as plsc

B = 16384
ED = 64
CF = 64
PF = 32
NW = 32
BPW = B // NW
H = BPW // 2


def _sc_gather(ct4, pt4, cid_hbm, pid_hbm, ce4_hbm, pe4_hbm,
               cidx_v, pidx_v, cbuf, pbuf, sem_c, sem_p):
    wid = lax.axis_index("s") * 2 + lax.axis_index("c")
    base = wid * BPW
    pltpu.sync_copy(cid_hbm.at[pl.ds(base, BPW)], cidx_v)
    pltpu.sync_copy(pid_hbm.at[pl.ds(base, BPW)], pidx_v)
    for h in range(2):
        cc = pltpu.async_copy(ct4.at[cidx_v.at[pl.ds(h * H, H)]], cbuf, sem_c)
        pc = pltpu.async_copy(pt4.at[pidx_v.at[pl.ds(h * H, H)]], pbuf, sem_p)
        cc.wait()
        pltpu.sync_copy(cbuf, ce4_hbm.at[pl.ds(base + h * H, H)])
        pc.wait()
        pltpu.sync_copy(pbuf, pe4_hbm.at[pl.ds(base + h * H, H)])


def _cvt_body(xT_ref, out_ref):
    u = lax.bitcast_convert_type(xT_ref[...], jnp.uint32)
    r = (u + jnp.uint32(0x7FFF) + ((u >> 16) & jnp.uint32(1))) >> 16
    w = r[:32, :] | (r[32:, :] << 16)
    q = w.shape[1] // 4
    parts = [jnp.transpose(w[:, j * q:(j + 1) * q]) for j in range(4)]
    out_ref[...] = lax.bitcast_convert_type(
        jnp.concatenate(parts, axis=1), jnp.int32)


def _pack_table(tbl):
    n = tbl.shape[0]
    bk = 16384
    grid = pl.cdiv(n, bk)
    return pl.pallas_call(
        _cvt_body,
        grid=(grid,),
        in_specs=[pl.BlockSpec((ED, bk), lambda i: (0, i))],
        out_specs=pl.BlockSpec((bk // 4, 128), lambda i: (i, 0)),
        out_shape=jax.ShapeDtypeStruct((grid * (bk // 4), 128), jnp.int32),
    )(tbl.T)


def _pick(e32, m):
    lo = lax.bitcast_convert_type(e32 << 16, jnp.float32)
    hi = lax.bitcast_convert_type(e32 & jnp.int32(-65536), jnp.float32)

    def sel(x):
        a = jnp.where(m < 1, x[:, :32], x[:, 32:64])
        b = jnp.where(m < 3, x[:, 64:96], x[:, 96:])
        return jnp.where(m < 2, a, b)

    return sel(lo), sel(hi)


def _mlp_body(ce4, pe4, cm, pm, cf, pf, w1ce, w1co, w1pe, w1po,
              w1cf, w1pf, b1, w2, b2, w3, b3, w4, b4, out_ref):
    ce_e, ce_o = _pick(ce4[...], cm[...])
    pe_e, pe_o = _pick(pe4[...], pm[...])
    x = jnp.dot(ce_e, w1ce[...], preferred_element_type=jnp.float32)
    x += jnp.dot(ce_o, w1co[...], preferred_element_type=jnp.float32)
    x += jnp.dot(pe_e, w1pe[...], preferred_element_type=jnp.float32)
    x += jnp.dot(pe_o, w1po[...], preferred_element_type=jnp.float32)
    x += jnp.dot(cf[...], w1cf[...], preferred_element_type=jnp.float32)
    x += jnp.dot(pf[...], w1pf[...], preferred_element_type=jnp.float32)
    h = jax.nn.relu(x + b1[...])
    h = jax.nn.relu(jnp.dot(h, w2[...], preferred_element_type=jnp.float32)
                    + b2[...])
    h = jax.nn.relu(jnp.dot(h, w3[...], preferred_element_type=jnp.float32)
                    + b3[...])
    o = jnp.dot(h, w4[...], preferred_element_type=jnp.float32) + b4[...]
    out_ref[...] = jax.nn.sigmoid(o)


def kernel(customer_ids, product_ids, customer_features, product_features,
           customer_table, product_table,
           W1, b1, g1, beta1, W2, b2, g2, beta2, W3, b3, g3, beta3, W4, b4):
    cid = customer_ids.astype(jnp.int32)
    pid = product_ids.astype(jnp.int32)
    nc = customer_table.shape[0]
    np_ = product_table.shape[0]
    ct4 = _pack_table(customer_table)
    pt4 = _pack_table(product_table)

    mesh = plsc.VectorSubcoreMesh(core_axis_name="c", subcore_axis_name="s")
    gather = pl.kernel(
        _sc_gather,
        out_type=(jax.ShapeDtypeStruct((B, 128), jnp.int32),
                  jax.ShapeDtypeStruct((B, 128), jnp.int32)),
        mesh=mesh,
        scratch_types=[
            pltpu.VMEM((BPW,), jnp.int32),
            pltpu.VMEM((BPW,), jnp.int32),
            pltpu.VMEM((H, 128), jnp.int32),
            pltpu.VMEM((H, 128), jnp.int32),
            pltpu.SemaphoreType.DMA,
            pltpu.SemaphoreType.DMA,
        ],
    )
    cu = (cid // 16384) * 4096 + cid % 4096
    pu = (pid // 16384) * 4096 + pid % 4096
    ce4, pe4 = gather(ct4, pt4, cu, pu)
    cm = ((cid % 16384) // 4096).reshape(B, 1)
    pm = ((pid % 16384) // 4096).reshape(B, 1)

    inv = 1.0 / jnp.sqrt(1.0 + 1e-5)
    s1 = g1 * inv
    s2 = g2 * inv
    s3 = g3 * inv
    w1f = W1 * s1[:, None]
    b1f = (b1 * s1 + beta1).reshape(1, -1)
    w2f = (W2 * s2[:, None]).T
    b2f = (b2 * s2 + beta2).reshape(1, -1)
    w3f = (W3 * s3[:, None]).T
    b3f = (b3 * s3 + beta3).reshape(1, -1)
    w4t = W4.T
    b4r = b4.reshape(1, 1)
    w1c = w1f[:, :ED].T
    w1p = w1f[:, ED:2 * ED].T
    w1cf = w1f[:, 2 * ED:2 * ED + CF].T
    w1pf = w1f[:, 2 * ED + CF:].T
    w1ce, w1co = w1c[:32], w1c[32:]
    w1pe, w1po = w1p[:32], w1p[32:]

    BM = 4096
    grid = B // BM
    row = lambda i: (i, 0)
    row3 = lambda i: (i, 0, 0)
    full = lambda i: (0, 0)
    out = pl.pallas_call(
        _mlp_body,
        grid=(grid,),
        in_specs=[
            pl.BlockSpec((BM, 128), row),
            pl.BlockSpec((BM, 128), row),
            pl.BlockSpec((BM, 1), row),
            pl.BlockSpec((BM, 1), row),
            pl.BlockSpec((BM, CF), row),
            pl.BlockSpec((BM, PF), row),
            pl.BlockSpec((32, 256), full),
            pl.BlockSpec((32, 256), full),
            pl.BlockSpec((32, 256), full),
            pl.BlockSpec((32, 256), full),
            pl.BlockSpec((CF, 256), full),
            pl.BlockSpec((PF, 256), full),
            pl.BlockSpec((1, 256), full),
            pl.BlockSpec((256, 128), full),
            pl.BlockSpec((1, 128), full),
            pl.BlockSpec((128, 64), full),
            pl.BlockSpec((1, 64), full),
            pl.BlockSpec((64, 1), full),
            pl.BlockSpec((1, 1), full),
        ],
        out_specs=pl.BlockSpec((BM, 1), row),
        out_shape=jax.ShapeDtypeStruct((B, 1), jnp.float32),
    )(ce4, pe4, cm, pm, customer_features, product_features,
      w1ce, w1co, w1pe, w1po, w1cf, w1pf, b1f, w2f, b2f, w3f, b3f, w4t, b4r)
    return out


# speedup vs baseline: 1.9483x; 1.0120x over previous
import jax
import jax.numpy as jnp
from jax import lax
from jax.experimental import pallas as pl
from jax.experimental.pallas import tpu as pltpu
from jax.experimental.pallas import tpu_sc as plsc

B = 16384
ED = 64
CF = 64
PF = 32
NW = 32
BPW = B // NW
H = BPW // 2


def _sc_gather(ct4, pt4, cid_hbm, pid_hbm, ce4_hbm, pe4_hbm,
               cidx_v, pidx_v, cbuf, pbuf, sem_c, sem_p):
    wid = lax.axis_index("s") * 2 + lax.axis_index("c")
    base = wid * BPW
    pltpu.sync_copy(cid_hbm.at[pl.ds(base, BPW)], cidx_v)
    pltpu.sync_copy(pid_hbm.at[pl.ds(base, BPW)], pidx_v)
    for h in range(2):
        cc = pltpu.async_copy(ct4.at[cidx_v.at[pl.ds(h * H, H)]], cbuf, sem_c)
        pc = pltpu.async_copy(pt4.at[pidx_v.at[pl.ds(h * H, H)]], pbuf, sem_p)
        cc.wait()
        pltpu.sync_copy(cbuf, ce4_hbm.at[pl.ds(base + h * H, H)])
        pc.wait()
        pltpu.sync_copy(pbuf, pe4_hbm.at[pl.ds(base + h * H, H)])


def _cvt_body(xT_ref, out_ref):
    u = lax.bitcast_convert_type(xT_ref[...], jnp.uint32)
    r = (u + jnp.uint32(0x7FFF) + ((u >> 16) & jnp.uint32(1))) >> 16
    w = r[:32, :] | (r[32:, :] << 16)
    q = w.shape[1] // 4
    parts = [jnp.transpose(w[:, j * q:(j + 1) * q]) for j in range(4)]
    out_ref[...] = lax.bitcast_convert_type(
        jnp.concatenate(parts, axis=1), jnp.int32)


def _pack_table(tbl):
    n = tbl.shape[0]
    bk = 16384
    grid = pl.cdiv(n, bk)
    return pl.pallas_call(
        _cvt_body,
        grid=(grid,),
        in_specs=[pl.BlockSpec((ED, bk), lambda i: (0, i))],
        out_specs=pl.BlockSpec((bk // 4, 128), lambda i: (i, 0)),
        out_shape=jax.ShapeDtypeStruct((grid * (bk // 4), 128), jnp.int32),
    )(tbl.T)


def _pick(e32, m):
    lo = lax.bitcast_convert_type(e32 << 16, jnp.float32)
    hi = lax.bitcast_convert_type(e32 & jnp.int32(-65536), jnp.float32)

    def sel(x):
        a = jnp.where(m < 1, x[:, :32], x[:, 32:64])
        b = jnp.where(m < 3, x[:, 64:96], x[:, 96:])
        return jnp.where(m < 2, a, b)

    return sel(lo), sel(hi)


def _mlp_body(ce4, pe4, cm, pm, cf, pf, w1ce, w1co, w1pe, w1po,
              w1cf, w1pf, b1, w2, b2, w3, b3, w4, b4, out_ref):
    ce_e, ce_o = _pick(ce4[...], jnp.transpose(cm[...]))
    pe_e, pe_o = _pick(pe4[...], jnp.transpose(pm[...]))
    x = jnp.dot(ce_e, w1ce[...], preferred_element_type=jnp.float32)
    x += jnp.dot(ce_o, w1co[...], preferred_element_type=jnp.float32)
    x += jnp.dot(pe_e, w1pe[...], preferred_element_type=jnp.float32)
    x += jnp.dot(pe_o, w1po[...], preferred_element_type=jnp.float32)
    x += jnp.dot(cf[...], w1cf[...], preferred_element_type=jnp.float32)
    x += jnp.dot(pf[...], w1pf[...], preferred_element_type=jnp.float32)
    h = jax.nn.relu(x + b1[...])
    h = jax.nn.relu(jnp.dot(h, w2[...], preferred_element_type=jnp.float32)
                    + b2[...])
    h = jax.nn.relu(jnp.dot(h, w3[...], preferred_element_type=jnp.float32)
                    + b3[...])
    o = jnp.dot(h, w4[...], preferred_element_type=jnp.float32) + b4[...]
    out_ref[...] = jax.nn.sigmoid(o)


def kernel(customer_ids, product_ids, customer_features, product_features,
           customer_table, product_table,
           W1, b1, g1, beta1, W2, b2, g2, beta2, W3, b3, g3, beta3, W4, b4):
    cid = customer_ids.astype(jnp.int32)
    pid = product_ids.astype(jnp.int32)
    nc = customer_table.shape[0]
    np_ = product_table.shape[0]
    ct4 = _pack_table(customer_table)
    pt4 = _pack_table(product_table)

    mesh = plsc.VectorSubcoreMesh(core_axis_name="c", subcore_axis_name="s")
    gather = pl.kernel(
        _sc_gather,
        out_type=(jax.ShapeDtypeStruct((B, 128), jnp.int32),
                  jax.ShapeDtypeStruct((B, 128), jnp.int32)),
        mesh=mesh,
        scratch_types=[
            pltpu.VMEM((BPW,), jnp.int32),
            pltpu.VMEM((BPW,), jnp.int32),
            pltpu.VMEM((H, 128), jnp.int32),
            pltpu.VMEM((H, 128), jnp.int32),
            pltpu.SemaphoreType.DMA,
            pltpu.SemaphoreType.DMA,
        ],
    )
    cu = (cid // 16384) * 4096 + cid % 4096
    pu = (pid // 16384) * 4096 + pid % 4096
    ce4, pe4 = gather(ct4, pt4, cu, pu)
    cm = ((cid % 16384) // 4096).reshape(1, B)
    pm = ((pid % 16384) // 4096).reshape(1, B)

    inv = 1.0 / jnp.sqrt(1.0 + 1e-5)
    s1 = g1 * inv
    s2 = g2 * inv
    s3 = g3 * inv
    w1f = W1 * s1[:, None]
    b1f = (b1 * s1 + beta1).reshape(1, -1)
    w2f = (W2 * s2[:, None]).T
    b2f = (b2 * s2 + beta2).reshape(1, -1)
    w3f = (W3 * s3[:, None]).T
    b3f = (b3 * s3 + beta3).reshape(1, -1)
    w4t = W4.T
    b4r = b4.reshape(1, 1)
    w1c = w1f[:, :ED].T
    w1p = w1f[:, ED:2 * ED].T
    w1cf = w1f[:, 2 * ED:2 * ED + CF].T
    w1pf = w1f[:, 2 * ED + CF:].T
    w1ce, w1co = w1c[:32], w1c[32:]
    w1pe, w1po = w1p[:32], w1p[32:]

    BM = 4096
    grid = B // BM
    row = lambda i: (i, 0)
    row3 = lambda i: (i, 0, 0)
    full = lambda i: (0, 0)
    out = pl.pallas_call(
        _mlp_body,
        grid=(grid,),
        in_specs=[
            pl.BlockSpec((BM, 128), row),
            pl.BlockSpec((BM, 128), row),
            pl.BlockSpec((1, BM), lambda i: (0, i)),
            pl.BlockSpec((1, BM), lambda i: (0, i)),
            pl.BlockSpec((BM, CF), row),
            pl.BlockSpec((BM, PF), row),
            pl.BlockSpec((32, 256), full),
            pl.BlockSpec((32, 256), full),
            pl.BlockSpec((32, 256), full),
            pl.BlockSpec((32, 256), full),
            pl.BlockSpec((CF, 256), full),
            pl.BlockSpec((PF, 256), full),
            pl.BlockSpec((1, 256), full),
            pl.BlockSpec((256, 128), full),
            pl.BlockSpec((1, 128), full),
            pl.BlockSpec((128, 64), full),
            pl.BlockSpec((1, 64), full),
            pl.BlockSpec((64, 1), full),
            pl.BlockSpec((1, 1), full),
        ],
        out_specs=pl.BlockSpec((BM, 1), row),
        out_shape=jax.ShapeDtypeStruct((B, 1), jnp.float32),
    )(ce4, pe4, cm, pm, customer_features, product_features,
      w1ce, w1co, w1pe, w1po, w1cf, w1pf, b1f, w2f, b2f, w3f, b3f, w4t, b4r)
    return out


# speedup vs baseline: 1.9614x; 1.0067x over previous
import jax
import jax.numpy as jnp
from jax import lax
from jax.experimental import pallas as pl
from jax.experimental.pallas import tpu as pltpu
from jax.experimental.pallas import tpu_sc as plsc

B = 16384
ED = 64
CF = 64
PF = 32
NW = 32
BPW = B // NW
H = BPW // 2


def _sc_gather1(t4, id_hbm, out_hbm, idx_v, buf, sem):
    wid = lax.axis_index("s") * 2 + lax.axis_index("c")
    base = wid * BPW
    pltpu.sync_copy(id_hbm.at[pl.ds(base, BPW)], idx_v)
    pltpu.async_copy(t4.at[idx_v], buf, sem).wait()
    pltpu.sync_copy(buf, out_hbm.at[pl.ds(base, BPW)])


def _cvt_body(xT_ref, out_ref):
    u = lax.bitcast_convert_type(xT_ref[...], jnp.uint32)
    r = (u + jnp.uint32(0x7FFF) + ((u >> 16) & jnp.uint32(1))) >> 16
    w = r[:32, :] | (r[32:, :] << 16)
    q = w.shape[1] // 4
    parts = [jnp.transpose(w[:, j * q:(j + 1) * q]) for j in range(4)]
    out_ref[...] = lax.bitcast_convert_type(
        jnp.concatenate(parts, axis=1), jnp.int32)


def _pack_table(tbl):
    n = tbl.shape[0]
    bk = 16384
    grid = pl.cdiv(n, bk)
    return pl.pallas_call(
        _cvt_body,
        grid=(grid,),
        in_specs=[pl.BlockSpec((ED, bk), lambda i: (0, i))],
        out_specs=pl.BlockSpec((bk // 4, 128), lambda i: (i, 0)),
        out_shape=jax.ShapeDtypeStruct((grid * (bk // 4), 128), jnp.int32),
    )(tbl.T)


def _pick(e32, m):
    lo = lax.bitcast_convert_type(e32 << 16, jnp.float32)
    hi = lax.bitcast_convert_type(e32 & jnp.int32(-65536), jnp.float32)

    def sel(x):
        a = jnp.where(m < 1, x[:, :32], x[:, 32:64])
        b = jnp.where(m < 3, x[:, 64:96], x[:, 96:])
        return jnp.where(m < 2, a, b)

    return sel(lo), sel(hi)


def _mlp_body(ce4, pe4, cm, pm, cf, pf, w1ce, w1co, w1pe, w1po,
              w1cf, w1pf, b1, w2, b2, w3, b3, w4, b4, out_ref):
    ce_e, ce_o = _pick(ce4[...], jnp.transpose(cm[...]))
    pe_e, pe_o = _pick(pe4[...], jnp.transpose(pm[...]))
    x = jnp.dot(ce_e, w1ce[...], preferred_element_type=jnp.float32)
    x += jnp.dot(ce_o, w1co[...], preferred_element_type=jnp.float32)
    x += jnp.dot(pe_e, w1pe[...], preferred_element_type=jnp.float32)
    x += jnp.dot(pe_o, w1po[...], preferred_element_type=jnp.float32)
    x += jnp.dot(cf[...], w1cf[...], preferred_element_type=jnp.float32)
    x += jnp.dot(pf[...], w1pf[...], preferred_element_type=jnp.float32)
    h = jax.nn.relu(x + b1[...])
    h = jax.nn.relu(jnp.dot(h, w2[...], preferred_element_type=jnp.float32)
                    + b2[...])
    h = jax.nn.relu(jnp.dot(h, w3[...], preferred_element_type=jnp.float32)
                    + b3[...])
    o = jnp.dot(h, w4[...], preferred_element_type=jnp.float32) + b4[...]
    out_ref[...] = jax.nn.sigmoid(o)


def kernel(customer_ids, product_ids, customer_features, product_features,
           customer_table, product_table,
           W1, b1, g1, beta1, W2, b2, g2, beta2, W3, b3, g3, beta3, W4, b4):
    cid = customer_ids.astype(jnp.int32)
    pid = product_ids.astype(jnp.int32)
    nc = customer_table.shape[0]
    np_ = product_table.shape[0]
    pt4 = _pack_table(product_table)
    ct4 = _pack_table(customer_table)

    mesh = plsc.VectorSubcoreMesh(core_axis_name="c", subcore_axis_name="s")
    gather = pl.kernel(
        _sc_gather1,
        out_type=jax.ShapeDtypeStruct((B, 128), jnp.int32),
        mesh=mesh,
        scratch_types=[
            pltpu.VMEM((BPW,), jnp.int32),
            pltpu.VMEM((BPW, 128), jnp.int32),
            pltpu.SemaphoreType.DMA,
        ],
    )
    cu = (cid // 16384) * 4096 + cid % 4096
    pu = (pid // 16384) * 4096 + pid % 4096
    pe4 = gather(pt4, pu)
    ce4 = gather(ct4, cu)
    cm = ((cid % 16384) // 4096).reshape(1, B)
    pm = ((pid % 16384) // 4096).reshape(1, B)

    inv = 1.0 / jnp.sqrt(1.0 + 1e-5)
    s1 = g1 * inv
    s2 = g2 * inv
    s3 = g3 * inv
    w1f = W1 * s1[:, None]
    b1f = (b1 * s1 + beta1).reshape(1, -1)
    w2f = (W2 * s2[:, None]).T
    b2f = (b2 * s2 + beta2).reshape(1, -1)
    w3f = (W3 * s3[:, None]).T
    b3f = (b3 * s3 + beta3).reshape(1, -1)
    w4t = W4.T
    b4r = b4.reshape(1, 1)
    w1c = w1f[:, :ED].T
    w1p = w1f[:, ED:2 * ED].T
    w1cf = w1f[:, 2 * ED:2 * ED + CF].T
    w1pf = w1f[:, 2 * ED + CF:].T
    w1ce, w1co = w1c[:32], w1c[32:]
    w1pe, w1po = w1p[:32], w1p[32:]

    BM = 4096
    grid = B // BM
    row = lambda i: (i, 0)
    row3 = lambda i: (i, 0, 0)
    full = lambda i: (0, 0)
    out = pl.pallas_call(
        _mlp_body,
        grid=(grid,),
        in_specs=[
            pl.BlockSpec((BM, 128), row),
            pl.BlockSpec((BM, 128), row),
            pl.BlockSpec((1, BM), lambda i: (0, i)),
            pl.BlockSpec((1, BM), lambda i: (0, i)),
            pl.BlockSpec((BM, CF), row),
            pl.BlockSpec((BM, PF), row),
            pl.BlockSpec((32, 256), full),
            pl.BlockSpec((32, 256), full),
            pl.BlockSpec((32, 256), full),
            pl.BlockSpec((32, 256), full),
            pl.BlockSpec((CF, 256), full),
            pl.BlockSpec((PF, 256), full),
            pl.BlockSpec((1, 256), full),
            pl.BlockSpec((256, 128), full),
            pl.BlockSpec((1, 128), full),
            pl.BlockSpec((128, 64), full),
            pl.BlockSpec((1, 64), full),
            pl.BlockSpec((64, 1), full),
            pl.BlockSpec((1, 1), full),
        ],
        out_specs=pl.BlockSpec((BM, 1), row),
        out_shape=jax.ShapeDtypeStruct((B, 1), jnp.float32),
    )(ce4, pe4, cm, pm, customer_features, product_features,
      w1ce, w1co, w1pe, w1po, w1cf, w1pf, b1f, w2f, b2f, w3f, b3f, w4t, b4r)
    return out


# speedup vs baseline: 1.9644x; 1.0015x over previous
import jax
import jax.numpy as jnp
from jax import lax
from jax.experimental import pallas as pl
from jax.experimental.pallas import tpu as pltpu
from jax.experimental.pallas import tpu_sc as plsc

B = 16384
ED = 64
CF = 64
PF = 32
NW = 32
BPW = B // NW


def _sc_gather1(t4, id_hbm, out_hbm, idx_v, buf, sem):
    wid = lax.axis_index("s") * 2 + lax.axis_index("c")
    base = wid * BPW
    pltpu.sync_copy(id_hbm.at[pl.ds(base, BPW)], idx_v)
    pltpu.async_copy(t4.at[idx_v], buf, sem).wait()
    pltpu.sync_copy(buf, out_hbm.at[pl.ds(base, BPW)])


def _cvt_body(xT_ref, out_ref):
    u = lax.bitcast_convert_type(xT_ref[...], jnp.uint32)
    r = (u + jnp.uint32(0x7FFF) + ((u >> 16) & jnp.uint32(1))) >> 16
    w = r[:32, :] | (r[32:, :] << 16)
    q = w.shape[1] // 4
    parts = [jnp.transpose(w[:, j * q:(j + 1) * q]) for j in range(4)]
    out_ref[...] = lax.bitcast_convert_type(
        jnp.concatenate(parts, axis=1), jnp.int32)


def _pack_table(tbl):
    n = tbl.shape[0]
    bk = 16384
    grid = pl.cdiv(n, bk)
    return pl.pallas_call(
        _cvt_body,
        grid=(grid,),
        in_specs=[pl.BlockSpec((ED, bk), lambda i: (0, i))],
        out_specs=pl.BlockSpec((bk // 4, 128), lambda i: (i, 0)),
        out_shape=jax.ShapeDtypeStruct((grid * (bk // 4), 128), jnp.int32),
    )(tbl.T)


def _pick(e32, m):
    lo = lax.bitcast_convert_type(e32 << 16, jnp.float32)
    hi = lax.bitcast_convert_type(e32 & jnp.int32(-65536), jnp.float32)

    def sel(x):
        a = jnp.where(m < 1, x[:, :32], x[:, 32:64])
        b = jnp.where(m < 3, x[:, 64:96], x[:, 96:])
        return jnp.where(m < 2, a, b)

    return sel(lo), sel(hi)


def _mlp_body(ce4, pe4, cm, pm, cf, pf, w1ce, w1co, w1pe, w1po,
              w1cf, w1pf, b1, w2, b2, w3, b3, w4, b4, out_ref):
    ce_e, ce_o = _pick(ce4[...], jnp.transpose(cm[...]))
    pe_e, pe_o = _pick(pe4[...], jnp.transpose(pm[...]))
    x = jnp.dot(ce_e, w1ce[...], preferred_element_type=jnp.float32)
    x += jnp.dot(ce_o, w1co[...], preferred_element_type=jnp.float32)
    x += jnp.dot(pe_e, w1pe[...], preferred_element_type=jnp.float32)
    x += jnp.dot(pe_o, w1po[...], preferred_element_type=jnp.float32)
    x += jnp.dot(cf[...], w1cf[...], preferred_element_type=jnp.float32)
    x += jnp.dot(pf[...], w1pf[...], preferred_element_type=jnp.float32)
    h = jax.nn.relu(x + b1[...])
    h = jax.nn.relu(jnp.dot(h, w2[...], preferred_element_type=jnp.float32)
                    + b2[...])
    h = jax.nn.relu(jnp.dot(h, w3[...], preferred_element_type=jnp.float32)
                    + b3[...])
    o = jnp.dot(h, w4[...], preferred_element_type=jnp.float32) + b4[...]
    out_ref[...] = jax.nn.sigmoid(o)


def kernel(customer_ids, product_ids, customer_features, product_features,
           customer_table, product_table,
           W1, b1, g1, beta1, W2, b2, g2, beta2, W3, b3, g3, beta3, W4, b4):
    cid = customer_ids.astype(jnp.int32)
    pid = product_ids.astype(jnp.int32)
    nc = customer_table.shape[0]
    np_ = product_table.shape[0]
    pt4 = _pack_table(product_table)
    ct4 = _pack_table(customer_table)

    mesh = plsc.VectorSubcoreMesh(core_axis_name="c", subcore_axis_name="s")
    gather = pl.kernel(
        _sc_gather1,
        out_type=jax.ShapeDtypeStruct((B, 128), jnp.int32),
        mesh=mesh,
        scratch_types=[
            pltpu.VMEM((BPW,), jnp.int32),
            pltpu.VMEM((BPW, 128), jnp.int32),
            pltpu.SemaphoreType.DMA,
        ],
    )
    cu = (cid // 16384) * 4096 + cid % 4096
    pu = (pid // 16384) * 4096 + pid % 4096
    pe4 = gather(pt4, pu)
    ce4 = gather(ct4, cu)
    cm = ((cid % 16384) // 4096).reshape(1, B)
    pm = ((pid % 16384) // 4096).reshape(1, B)

    inv = 1.0 / jnp.sqrt(1.0 + 1e-5)
    s1 = g1 * inv
    s2 = g2 * inv
    s3 = g3 * inv
    w1f = W1 * s1[:, None]
    b1f = (b1 * s1 + beta1).reshape(1, -1)
    w2f = (W2 * s2[:, None]).T
    b2f = (b2 * s2 + beta2).reshape(1, -1)
    w3f = (W3 * s3[:, None]).T
    b3f = (b3 * s3 + beta3).reshape(1, -1)
    w4t = W4.T
    b4r = b4.reshape(1, 1)
    w1c = w1f[:, :ED].T
    w1p = w1f[:, ED:2 * ED].T
    w1cf = w1f[:, 2 * ED:2 * ED + CF].T
    w1pf = w1f[:, 2 * ED + CF:].T
    w1ce, w1co = w1c[:32], w1c[32:]
    w1pe, w1po = w1p[:32], w1p[32:]

    BM = 4096
    grid = B // BM
    row = lambda i: (i, 0)
    row3 = lambda i: (i, 0, 0)
    full = lambda i: (0, 0)
    out = pl.pallas_call(
        _mlp_body,
        grid=(grid,),
        in_specs=[
            pl.BlockSpec((BM, 128), row),
            pl.BlockSpec((BM, 128), row),
            pl.BlockSpec((1, BM), lambda i: (0, i)),
            pl.BlockSpec((1, BM), lambda i: (0, i)),
            pl.BlockSpec((BM, CF), row),
            pl.BlockSpec((BM, PF), row),
            pl.BlockSpec((32, 256), full),
            pl.BlockSpec((32, 256), full),
            pl.BlockSpec((32, 256), full),
            pl.BlockSpec((32, 256), full),
            pl.BlockSpec((CF, 256), full),
            pl.BlockSpec((PF, 256), full),
            pl.BlockSpec((1, 256), full),
            pl.BlockSpec((256, 128), full),
            pl.BlockSpec((1, 128), full),
            pl.BlockSpec((128, 64), full),
            pl.BlockSpec((1, 64), full),
            pl.BlockSpec((64, 1), full),
            pl.BlockSpec((1, 1), full),
        ],
        out_specs=pl.BlockSpec((BM, 1), row),
        out_shape=jax.ShapeDtypeStruct((B, 1), jnp.float32),
    )(ce4, pe4, cm, pm, customer_features, product_features,
      w1ce, w1co, w1pe, w1po, w1cf, w1pf, b1f, w2f, b2f, w3f, b3f, w4t, b4r)
    return out
